# baseline (speedup 1.0000x reference)
import jax
import jax.numpy as jnp
from jax import lax
from jax.experimental import pallas as pl
from jax.experimental.pallas import tpu as pltpu

N = 8192
E = 8
D_IN = 1024
D_OUT = 1024
TM = 256
KB = E * D_IN + 128


def _moe_tile(g_ref, x_ref, wb_ref, out_ref, z_ref):
    g = g_ref[...]

    neg_inf = jnp.float32(-jnp.inf)
    m1 = jnp.full((TM, 1), neg_inf, jnp.float32)
    i1 = jnp.zeros((TM, 1), jnp.int32)
    for e in range(E):
        ge = g[:, e : e + 1]
        better = ge > m1
        m1 = jnp.where(better, ge, m1)
        i1 = jnp.where(better, e, i1)
    m2 = jnp.full((TM, 1), neg_inf, jnp.float32)
    i2 = jnp.zeros((TM, 1), jnp.int32)
    for e in range(E):
        ge = jnp.where(i1 == e, neg_inf, g[:, e : e + 1])
        better = ge > m2
        m2 = jnp.where(better, ge, m2)
        i2 = jnp.where(better, e, i2)
    p1 = 1.0 / (1.0 + jnp.exp(m2 - m1))
    p2 = 1.0 - p1

    x = x_ref[...]
    zb = jnp.zeros((TM, 128), jnp.float32)
    col = lax.broadcasted_iota(jnp.int32, (TM, 128), 1)
    for e in range(E):
        se = jnp.where(i1 == e, p1, 0.0) + jnp.where(i2 == e, p2, 0.0)
        z_ref[:, e * D_IN : (e + 1) * D_IN] = x * se.astype(jnp.bfloat16)
        zb = jnp.where(col == e, se, zb)
    z_ref[:, E * D_IN :] = zb.astype(jnp.bfloat16)

    out_ref[...] = lax.dot_general(
        z_ref[...],
        wb_ref[...],
        (((1,), (0,)), ((), ())),
        preferred_element_type=jnp.float32,
    )


@jax.jit
def kernel(X, G, W, b):
    Xb = X.astype(jnp.bfloat16)
    Wrows = jnp.swapaxes(W, 1, 2).reshape(E * D_IN, D_OUT)
    Brows = jnp.zeros((128, D_OUT), jnp.float32).at[:E].set(b)
    Wbig = jnp.concatenate([Wrows, Brows], axis=0).astype(jnp.bfloat16)
    grid = (N // TM,)
    return pl.pallas_call(
        _moe_tile,
        grid=grid,
        in_specs=[
            pl.BlockSpec((TM, E), lambda i: (i, 0)),
            pl.BlockSpec((TM, D_IN), lambda i: (i, 0)),
            pl.BlockSpec((KB, D_OUT), lambda i: (0, 0)),
        ],
        out_specs=pl.BlockSpec((TM, D_OUT), lambda i: (i, 0)),
        out_shape=jax.ShapeDtypeStruct((N, D_OUT), jnp.float32),
        scratch_shapes=[pltpu.VMEM((TM, KB), jnp.bfloat16)],
        compiler_params=pltpu.CompilerParams(
            dimension_semantics=("arbitrary",),
        ),
    )(G, Xb, Wbig)
